# baseline (speedup 1.0000x reference)
import functools

import jax
import jax.numpy as jnp
from jax import lax
from jax.experimental import pallas as pl
from jax.experimental.pallas import tpu as pltpu
from jax.experimental.pallas import tpu_sc as plsc

_NUM_FIELDS = 26
_VOCAB = 100000
_EMBED_DIM = 32
_BATCH = 4096

_NC = 2
_NS = 16

_mesh = plsc.VectorSubcoreMesh(core_axis_name="c", subcore_axis_name="s")


@functools.partial(
    pl.kernel,
    mesh=_mesh,
    out_type=jax.ShapeDtypeStruct((_NUM_FIELDS * _EMBED_DIM, _BATCH), jnp.float32),
    scratch_types=[
        pltpu.VMEM((2 * _BATCH,), jnp.int32),
        pltpu.VMEM((_VOCAB,), jnp.float32),
        pltpu.VMEM((2 * _BATCH,), jnp.float32),
        pltpu.VMEM((_NUM_FIELDS * 32,), jnp.float32),
        pltpu.SemaphoreType.DMA,
        pltpu.SemaphoreType.DMA,
        pltpu.SemaphoreType.DMA,
        pltpu.SemaphoreType.DMA,
        pltpu.SemaphoreType.DMA,
    ],
    compiler_params=pltpu.CompilerParams(
        use_tc_tiling_on_sc=True, needs_layout_passes=False
    ),
)
def _embed_gather(
    x_hbm, tab_hbm, tails_hbm, out_hbm,
    idx_v, vec_v, out_v, tail_v,
    sem_idx, sem_va, sem_vb, sem_tail, sem_out,
):
    wid = lax.axis_index("s") * _NC + lax.axis_index("c")
    nvec = _BATCH // 16
    h0 = 50048
    hb = 99968

    def copy_a(f):
        return pltpu.async_copy(
            tab_hbm.at[f, wid, pl.ds(0, h0)], vec_v.at[pl.ds(0, h0)], sem_va
        )

    def copy_b(f):
        return pltpu.async_copy(
            tab_hbm.at[f, wid, pl.ds(h0, hb - h0)], vec_v.at[pl.ds(h0, hb - h0)], sem_vb
        )

    idx_cp = [None] * _NUM_FIELDS
    a_cp = [None] * _NUM_FIELDS
    b_cp = [None] * _NUM_FIELDS
    out_cp = [None] * _NUM_FIELDS
    idx_cp[0] = pltpu.async_copy(x_hbm.at[0], idx_v.at[pl.ds(0, _BATCH)], sem_idx)
    a_cp[0] = copy_a(0)
    b_cp[0] = copy_b(0)
    pltpu.async_copy(tails_hbm.at[wid], tail_v, sem_tail).wait()

    for f in range(_NUM_FIELDS):
        p = f % 2
        ib = p * _BATCH
        ob = p * _BATCH
        a_cp[f].wait()
        idx_cp[f].wait()
        if f + 1 < _NUM_FIELDS:
            idx_cp[f + 1] = pltpu.async_copy(
                x_hbm.at[f + 1], idx_v.at[pl.ds((1 - p) * _BATCH, _BATCH)], sem_idx
            )

        @plsc.parallel_loop(0, nvec, unroll=8)
        def _(t):
            ids = idx_v[pl.ds(ib + t * 16, 16)]
            m = ids < h0
            ga = plsc.load_gather(vec_v, [jnp.minimum(ids, h0 - 1)], mask=m)
            out_v[pl.ds(ob + t * 16, 16)] = jnp.where(m, ga, 0.0)

        b_cp[f].wait()
        if f + 1 < _NUM_FIELDS:
            a_cp[f + 1] = copy_a(f + 1)

        vec_v[pl.ds(hb, 16)] = tail_v[pl.ds(f * 32, 16)]
        vec_v[pl.ds(hb + 16, 16)] = tail_v[pl.ds(f * 32 + 16, 16)]

        @plsc.parallel_loop(0, nvec, unroll=8)
        def _(t):
            ids = idx_v[pl.ds(ib + t * 16, 16)]
            m = ids >= h0
            gb = plsc.load_gather(vec_v, [jnp.maximum(ids, h0)], mask=m)
            plsc.addupdate(out_v.at[pl.ds(ob + t * 16, 16)], jnp.where(m, gb, 0.0))

        if f + 1 < _NUM_FIELDS:
            b_cp[f + 1] = copy_b(f + 1)
        if f > 0:
            out_cp[f - 1].wait()
        out_cp[f] = pltpu.async_copy(
            out_v.at[pl.ds(ob, _BATCH)], out_hbm.at[f * _EMBED_DIM + wid], sem_out
        )

    out_cp[_NUM_FIELDS - 1].wait()


def kernel(x, tables):
    x_t = x.astype(jnp.int32).T
    tab_t = jnp.transpose(tables, (0, 2, 1))
    tails = jnp.transpose(tables[:, _VOCAB - 32 :, :], (2, 0, 1)).reshape(
        _EMBED_DIM, _NUM_FIELDS * 32
    )
    out_t = _embed_gather(x_t, tab_t, tails)
    return out_t.T


# speedup vs baseline: 1.0063x; 1.0063x over previous
import functools

import jax
import jax.numpy as jnp
from jax import lax
from jax.experimental import pallas as pl
from jax.experimental.pallas import tpu as pltpu
from jax.experimental.pallas import tpu_sc as plsc

_NUM_FIELDS = 26
_VOCAB = 100000
_EMBED_DIM = 32
_BATCH = 4096

_NC = 2
_NS = 16

_mesh = plsc.VectorSubcoreMesh(core_axis_name="c", subcore_axis_name="s")


@functools.partial(
    pl.kernel,
    mesh=_mesh,
    out_type=jax.ShapeDtypeStruct((_NUM_FIELDS * _EMBED_DIM, _BATCH), jnp.float32),
    scratch_types=[
        pltpu.VMEM((2 * _BATCH,), jnp.int32),
        pltpu.VMEM((_VOCAB,), jnp.float32),
        pltpu.VMEM((2 * _BATCH,), jnp.float32),
        pltpu.VMEM((_NUM_FIELDS * 32,), jnp.float32),
        pltpu.SemaphoreType.DMA,
        pltpu.SemaphoreType.DMA,
        pltpu.SemaphoreType.DMA,
        pltpu.SemaphoreType.DMA,
        pltpu.SemaphoreType.DMA,
    ],
    compiler_params=pltpu.CompilerParams(
        use_tc_tiling_on_sc=True, needs_layout_passes=False
    ),
)
def _embed_gather(
    x_hbm, tab_hbm, tails_hbm, out_hbm,
    idx_v, vec_v, out_v, tail_v,
    sem_idx, sem_va, sem_vb, sem_tail, sem_out,
):
    wid = lax.axis_index("s") * _NC + lax.axis_index("c")
    nvec = _BATCH // 16
    h0 = 50048
    hb = 99968

    def copy_a(f):
        return pltpu.async_copy(
            tab_hbm.at[f, wid, pl.ds(0, h0)], vec_v.at[pl.ds(0, h0)], sem_va
        )

    def copy_b(f):
        return pltpu.async_copy(
            tab_hbm.at[f, wid, pl.ds(h0, hb - h0)], vec_v.at[pl.ds(h0, hb - h0)], sem_vb
        )

    idx_cp = [None] * _NUM_FIELDS
    a_cp = [None] * _NUM_FIELDS
    b_cp = [None] * _NUM_FIELDS
    out_cp = [None] * _NUM_FIELDS
    idx_cp[0] = pltpu.async_copy(x_hbm.at[0], idx_v.at[pl.ds(0, _BATCH)], sem_idx)
    a_cp[0] = copy_a(0)
    b_cp[0] = copy_b(0)
    pltpu.async_copy(tails_hbm.at[wid], tail_v, sem_tail).wait()

    for f in range(_NUM_FIELDS):
        p = f % 2
        ib = p * _BATCH
        ob = p * _BATCH
        a_cp[f].wait()
        idx_cp[f].wait()
        if f + 1 < _NUM_FIELDS:
            idx_cp[f + 1] = pltpu.async_copy(
                x_hbm.at[f + 1], idx_v.at[pl.ds((1 - p) * _BATCH, _BATCH)], sem_idx
            )

        @plsc.parallel_loop(0, nvec, unroll=4)
        def _(t):
            ids = idx_v[pl.ds(ib + t * 16, 16)]
            out_v[pl.ds(ob + t * 16, 16)] = plsc.load_gather(vec_v, [ids])

        b_cp[f].wait()
        if f + 1 < _NUM_FIELDS:
            a_cp[f + 1] = copy_a(f + 1)

        vec_v[pl.ds(hb, 16)] = tail_v[pl.ds(f * 32, 16)]
        vec_v[pl.ds(hb + 16, 16)] = tail_v[pl.ds(f * 32 + 16, 16)]

        @plsc.parallel_loop(0, nvec, unroll=4)
        def _(t):
            ids = idx_v[pl.ds(ib + t * 16, 16)]
            m = ids >= h0
            gb = plsc.load_gather(vec_v, [jnp.maximum(ids, h0)], mask=m)
            prev = out_v[pl.ds(ob + t * 16, 16)]
            out_v[pl.ds(ob + t * 16, 16)] = jnp.where(m, gb, prev)

        if f + 1 < _NUM_FIELDS:
            b_cp[f + 1] = copy_b(f + 1)
        if f > 0:
            out_cp[f - 1].wait()
        out_cp[f] = pltpu.async_copy(
            out_v.at[pl.ds(ob, _BATCH)], out_hbm.at[f * _EMBED_DIM + wid], sem_out
        )

    out_cp[_NUM_FIELDS - 1].wait()


def kernel(x, tables):
    x_t = x.astype(jnp.int32).T
    tab_t = jnp.transpose(tables, (0, 2, 1))
    tails = jnp.transpose(tables[:, _VOCAB - 32 :, :], (2, 0, 1)).reshape(
        _EMBED_DIM, _NUM_FIELDS * 32
    )
    out_t = _embed_gather(x_t, tab_t, tails)
    return out_t.T


# speedup vs baseline: 1.0399x; 1.0333x over previous
import functools

import jax
import jax.numpy as jnp
from jax import lax
from jax.experimental import pallas as pl
from jax.experimental.pallas import tpu as pltpu
from jax.experimental.pallas import tpu_sc as plsc

_NUM_FIELDS = 26
_VOCAB = 100000
_EMBED_DIM = 32
_BATCH = 4096

_NC = 2
_NS = 16

_mesh = plsc.VectorSubcoreMesh(core_axis_name="c", subcore_axis_name="s")


@functools.partial(
    pl.kernel,
    mesh=_mesh,
    out_type=jax.ShapeDtypeStruct((_NUM_FIELDS * _EMBED_DIM, _BATCH), jnp.float32),
    scratch_types=[
        pltpu.VMEM((2 * _BATCH,), jnp.int32),
        pltpu.VMEM((_VOCAB,), jnp.float32),
        pltpu.VMEM((2 * _BATCH,), jnp.float32),
        pltpu.VMEM((_NUM_FIELDS * 32,), jnp.float32),
        pltpu.SemaphoreType.DMA,
        pltpu.SemaphoreType.DMA,
        pltpu.SemaphoreType.DMA,
        pltpu.SemaphoreType.DMA,
        pltpu.SemaphoreType.DMA,
    ],
    compiler_params=pltpu.CompilerParams(
        use_tc_tiling_on_sc=True, needs_layout_passes=False
    ),
)
def _embed_gather(
    x_hbm, tab_hbm, tails_hbm, out_hbm,
    idx_v, vec_v, out_v, tail_v,
    sem_idx, sem_va, sem_vb, sem_tail, sem_out,
):
    wid = lax.axis_index("s") * _NC + lax.axis_index("c")
    nvec = _BATCH // 16
    h0 = 50048
    hb = 99968

    def copy_a(f):
        return pltpu.async_copy(
            tab_hbm.at[f, wid, pl.ds(0, h0)], vec_v.at[pl.ds(0, h0)], sem_va
        )

    def copy_b(f):
        return pltpu.async_copy(
            tab_hbm.at[f, wid, pl.ds(h0, hb - h0)], vec_v.at[pl.ds(h0, hb - h0)], sem_vb
        )

    idx_cp = [None] * _NUM_FIELDS
    a_cp = [None] * _NUM_FIELDS
    b_cp = [None] * _NUM_FIELDS
    out_cp = [None] * _NUM_FIELDS
    idx_cp[0] = pltpu.async_copy(x_hbm.at[0], idx_v.at[pl.ds(0, _BATCH)], sem_idx)
    a_cp[0] = copy_a(0)
    b_cp[0] = copy_b(0)
    pltpu.async_copy(tails_hbm.at[wid], tail_v, sem_tail).wait()

    for f in range(_NUM_FIELDS):
        p = f % 2
        ib = p * _BATCH
        ob = p * _BATCH
        a_cp[f].wait()
        idx_cp[f].wait()
        if f + 1 < _NUM_FIELDS:
            idx_cp[f + 1] = pltpu.async_copy(
                x_hbm.at[f + 1], idx_v.at[pl.ds((1 - p) * _BATCH, _BATCH)], sem_idx
            )

        @plsc.parallel_loop(0, nvec, unroll=4)
        def _(t):
            ids = idx_v[pl.ds(ib + t * 16, 16)]
            out_v[pl.ds(ob + t * 16, 16)] = plsc.load_gather(vec_v, [ids])

        if f + 1 < _NUM_FIELDS:
            a_cp[f + 1] = copy_a(f + 1)
        b_cp[f].wait()

        vec_v[pl.ds(hb, 16)] = tail_v[pl.ds(f * 32, 16)]
        vec_v[pl.ds(hb + 16, 16)] = tail_v[pl.ds(f * 32 + 16, 16)]

        @plsc.parallel_loop(0, nvec, unroll=4)
        def _(t):
            ids = idx_v[pl.ds(ib + t * 16, 16)]
            m = ids >= h0
            gb = plsc.load_gather(vec_v, [jnp.maximum(ids, h0)], mask=m)
            prev = out_v[pl.ds(ob + t * 16, 16)]
            out_v[pl.ds(ob + t * 16, 16)] = jnp.where(m, gb, prev)

        if f + 1 < _NUM_FIELDS:
            b_cp[f + 1] = copy_b(f + 1)
        if f > 0:
            out_cp[f - 1].wait()
        out_cp[f] = pltpu.async_copy(
            out_v.at[pl.ds(ob, _BATCH)], out_hbm.at[f * _EMBED_DIM + wid], sem_out
        )

    out_cp[_NUM_FIELDS - 1].wait()


def kernel(x, tables):
    x_t = x.astype(jnp.int32).T
    tab_t = jnp.transpose(tables, (0, 2, 1))
    tails = jnp.transpose(tables[:, _VOCAB - 32 :, :], (2, 0, 1)).reshape(
        _EMBED_DIM, _NUM_FIELDS * 32
    )
    out_t = _embed_gather(x_t, tab_t, tails)
    return out_t.T
